# baseline (speedup 1.0000x reference)
import jax
import jax.numpy as jnp
from jax.experimental import pallas as pl
from jax.experimental.pallas import tpu as pltpu


def kernel(x, row_w, col_w):
    n, dim, h, w = x.shape
    half = dim // 2
    hw = h * w
    wide = 4096
    dpr = wide // hw
    tile_rows = dim // dpr
    half_rows = tile_rows // 2
    rows = n * tile_rows

    def body(row_ref, col_ref, out_ref, buf, sem):
        lane = jax.lax.broadcasted_iota(jnp.int32, (w, hw), 1)
        src = jax.lax.broadcasted_iota(jnp.int32, (w, hw), 0)
        p = (lane % w == src).astype(jnp.float32)
        lane_h = jax.lax.broadcasted_iota(jnp.int32, (h, hw), 1)
        src_h = jax.lax.broadcasted_iota(jnp.int32, (h, hw), 0)
        q = (lane_h // w == src_h).astype(jnp.float32)
        xe = jax.lax.dot_general(
            col_ref[...], p, (((0,), (0,)), ((), ())),
            preferred_element_type=jnp.float32,
        )
        ye = jax.lax.dot_general(
            row_ref[...], q, (((0,), (0,)), ((), ())),
            preferred_element_type=jnp.float32,
        )
        bsel = jax.lax.broadcasted_iota(jnp.int32, (half, half_rows), 0)
        rsel = jax.lax.broadcasted_iota(jnp.int32, (half, half_rows), 1)
        for t in range(dpr):
            st = (bsel == dpr * rsel + t).astype(jnp.float32)
            blk_c = jax.lax.dot_general(
                st, xe, (((0,), (0,)), ((), ())),
                preferred_element_type=jnp.float32,
            )
            blk_r = jax.lax.dot_general(
                st, ye, (((0,), (0,)), ((), ())),
                preferred_element_type=jnp.float32,
            )
            for nn in range(n):
                base = nn * tile_rows
                buf[base:base + half_rows, t * hw:(t + 1) * hw] = blk_c
                buf[base + half_rows:base + tile_rows,
                    t * hw:(t + 1) * hw] = blk_r
        cp = pltpu.make_async_copy(buf, out_ref, sem)
        cp.start()
        cp.wait()

    out = pl.pallas_call(
        body,
        in_specs=[
            pl.BlockSpec(memory_space=pltpu.VMEM),
            pl.BlockSpec(memory_space=pltpu.VMEM),
        ],
        out_specs=pl.BlockSpec(memory_space=pl.ANY),
        out_shape=jax.ShapeDtypeStruct((rows, wide), jnp.float32),
        scratch_shapes=[
            pltpu.VMEM((rows, wide), jnp.float32),
            pltpu.SemaphoreType.DMA,
        ],
    )(row_w[:h], col_w[:w])
    return out.reshape(n, dim, h, w)


# speedup vs baseline: 3.5499x; 3.5499x over previous
import jax
import jax.numpy as jnp
from jax.experimental import pallas as pl
from jax.experimental.pallas import tpu as pltpu


def kernel(x, row_w, col_w):
    n, dim, h, w = x.shape
    half = dim // 2
    hw = h * w

    def body(row_ref, col_ref, out_ref, buf, sem):
        lane = jax.lax.broadcasted_iota(jnp.int32, (w, hw), 1)
        src = jax.lax.broadcasted_iota(jnp.int32, (w, hw), 0)
        p = (lane % w == src).astype(jnp.float32)
        lane_h = jax.lax.broadcasted_iota(jnp.int32, (h, hw), 1)
        src_h = jax.lax.broadcasted_iota(jnp.int32, (h, hw), 0)
        q = (lane_h // w == src_h).astype(jnp.float32)
        xe = jax.lax.dot_general(
            col_ref[...], p, (((0,), (0,)), ((), ())),
            preferred_element_type=jnp.float32,
        )
        ye = jax.lax.dot_general(
            row_ref[...], q, (((0,), (0,)), ((), ())),
            preferred_element_type=jnp.float32,
        )
        buf[0:half, :] = xe
        buf[half:dim, :] = ye
        copies = []
        for k in range(n):
            cp = pltpu.make_async_copy(buf, out_ref.at[k], sem.at[k])
            cp.start(priority=k % 2)
            copies.append(cp)
        for cp in copies:
            cp.wait()

    out = pl.pallas_call(
        body,
        in_specs=[
            pl.BlockSpec(memory_space=pltpu.VMEM),
            pl.BlockSpec(memory_space=pltpu.VMEM),
        ],
        out_specs=pl.BlockSpec(memory_space=pl.ANY),
        out_shape=jax.ShapeDtypeStruct((n, dim, hw), jnp.float32),
        scratch_shapes=[
            pltpu.VMEM((dim, hw), jnp.float32),
            pltpu.SemaphoreType.DMA((n,)),
        ],
    )(row_w[:h], col_w[:w])
    return out.reshape(n, dim, h, w)
